# baseline (speedup 1.0000x reference)
import functools

import jax
import jax.numpy as jnp
from jax import lax
from jax.experimental import pallas as pl
from jax.experimental.pallas import tpu as pltpu
from jax.experimental.pallas import tpu_sc as plsc

_Q = 4
_D = 128
_B = 16384
_NC = 2
_NS = 16
_NW = _NC * _NS
_RPW = _B // _NW
_VEC = 16
_CH = _RPW // 2


def _sc_body(x_hbm, lt_hbm, out_hbm, xbuf, tbuf, res, sem0, sem1):
    wid = lax.axis_index("s") * _NC + lax.axis_index("c")
    base = wid * _RPW
    pltpu.sync_copy(lt_hbm, tbuf)

    cp0 = pltpu.make_async_copy(
        x_hbm.at[pl.ds(base * _D, _CH * _D)],
        xbuf.at[pl.ds(0, _CH * _D)], sem0)
    cp1 = pltpu.make_async_copy(
        x_hbm.at[pl.ds((base + _CH) * _D, _CH * _D)],
        xbuf.at[pl.ds(_CH * _D, _CH * _D)], sem1)
    cp0.start()
    cp1.start()

    lanes = lax.iota(jnp.int32, _VEC)
    rows_off = lanes * _D

    _UNROLL = 16

    def blk_body(b, carry):
        @pl.when(b == 0)
        def _():
            cp0.wait()

        @pl.when(b == _CH // _VEC)
        def _():
            cp1.wait()

        rows_b = rows_off + b * (_VEC * _D)

        def chunk_steps(u, accs):
            accs = list(accs)
            c0 = (lanes + u * _UNROLL) & (_D - 1)
            cs = [(c0 + k) & (_D - 1) for k in range(4)]
            for t in range(_UNROLL):
                c = cs[t % 4]
                xv = plsc.load_gather(xbuf, [rows_b + c])
                accs[t % 8] = accs[t % 8] + plsc.load_gather(
                    tbuf, [xv * _D + c])
                cs[t % 4] = (c + 4) & (_D - 1)
            return tuple(accs)

        zero = jnp.zeros((_VEC,), jnp.float32)
        accs = lax.fori_loop(0, _D // _UNROLL, chunk_steps, (zero,) * 8)
        s01 = (accs[0] + accs[1]) + (accs[2] + accs[3])
        s23 = (accs[4] + accs[5]) + (accs[6] + accs[7])
        res[pl.ds(b * _VEC, _VEC)] = s01 + s23
        return carry

    lax.fori_loop(0, _RPW // _VEC, blk_body, 0)

    pltpu.sync_copy(res, out_hbm.at[pl.ds(base, _RPW)])


_sc_call = functools.partial(
    pl.kernel,
    out_type=jax.ShapeDtypeStruct((_B,), jnp.float32),
    mesh=plsc.VectorSubcoreMesh(core_axis_name="c", subcore_axis_name="s"),
    compiler_params=pltpu.CompilerParams(needs_layout_passes=False),
    scratch_types=[
        pltpu.VMEM((_RPW * _D,), jnp.int32),
        pltpu.VMEM((_D * _Q,), jnp.float32),
        pltpu.VMEM((_RPW,), jnp.float32),
        pltpu.SemaphoreType.DMA,
        pltpu.SemaphoreType.DMA,
    ],
)(_sc_body)


def kernel(x, category_parameters):
    lt = jnp.log(category_parameters * (1.0 - 0.2 * _Q) + 0.2)
    out = _sc_call(x.astype(jnp.int32).reshape(-1),
                   lt.reshape(-1).astype(jnp.float32))
    return lax.stop_gradient(out[:, None])


# speedup vs baseline: 1.0076x; 1.0076x over previous
import functools

import jax
import jax.numpy as jnp
from jax import lax
from jax.experimental import pallas as pl
from jax.experimental.pallas import tpu as pltpu
from jax.experimental.pallas import tpu_sc as plsc

_Q = 4
_D = 128
_B = 16384
_NC = 2
_NS = 16
_NW = _NC * _NS
_RPW = _B // _NW
_VEC = 16
_CH = _RPW // 2


def _sc_body(x_hbm, lt_hbm, out_hbm, xbuf, tbuf, res, sem0, sem1):
    wid = lax.axis_index("s") * _NC + lax.axis_index("c")
    base = wid * _RPW
    pltpu.sync_copy(lt_hbm, tbuf)

    cp0 = pltpu.make_async_copy(
        x_hbm.at[pl.ds(base * _D, _CH * _D)],
        xbuf.at[pl.ds(0, _CH * _D)], sem0)
    cp1 = pltpu.make_async_copy(
        x_hbm.at[pl.ds((base + _CH) * _D, _CH * _D)],
        xbuf.at[pl.ds(_CH * _D, _CH * _D)], sem1)
    cp0.start()
    cp1.start()

    lanes = lax.iota(jnp.int32, _VEC)
    rows_off = lanes * _D

    _UNROLL = 16

    def blk_body(b, carry):
        @pl.when(b == 0)
        def _():
            cp0.wait()

        @pl.when(b == _CH // _VEC)
        def _():
            cp1.wait()

        rows_b = rows_off + b * (_VEC * _D)

        def chunk_steps(u, accs):
            accs = list(accs)
            c0 = (lanes + u * _UNROLL) & (_D - 1)
            cs = [c0, (c0 + 1) & (_D - 1)]
            for t in range(_UNROLL):
                c = cs[t % 2]
                xv = plsc.load_gather(xbuf, [rows_b + c])
                accs[t % 4] = accs[t % 4] + plsc.load_gather(
                    tbuf, [xv * _D + c])
                cs[t % 2] = (c + 2) & (_D - 1)
            return tuple(accs)

        zero = jnp.zeros((_VEC,), jnp.float32)
        accs = lax.fori_loop(0, _D // _UNROLL, chunk_steps, (zero,) * 4)
        res[pl.ds(b * _VEC, _VEC)] = (accs[0] + accs[1]) + (accs[2] + accs[3])
        return carry

    lax.fori_loop(0, _RPW // _VEC, blk_body, 0)

    pltpu.sync_copy(res, out_hbm.at[pl.ds(base, _RPW)])


_sc_call = functools.partial(
    pl.kernel,
    out_type=jax.ShapeDtypeStruct((_B,), jnp.float32),
    mesh=plsc.VectorSubcoreMesh(core_axis_name="c", subcore_axis_name="s"),
    compiler_params=pltpu.CompilerParams(needs_layout_passes=False),
    scratch_types=[
        pltpu.VMEM((_RPW * _D,), jnp.int32),
        pltpu.VMEM((_D * _Q,), jnp.float32),
        pltpu.VMEM((_RPW,), jnp.float32),
        pltpu.SemaphoreType.DMA,
        pltpu.SemaphoreType.DMA,
    ],
)(_sc_body)


def kernel(x, category_parameters):
    lt = jnp.log(category_parameters * (1.0 - 0.2 * _Q) + 0.2)
    out = _sc_call(x.astype(jnp.int32).reshape(-1),
                   lt.reshape(-1).astype(jnp.float32))
    return lax.stop_gradient(out[:, None])


# speedup vs baseline: 1.0603x; 1.0523x over previous
import functools

import jax
import jax.numpy as jnp
from jax import lax
from jax.experimental import pallas as pl
from jax.experimental.pallas import tpu as pltpu
from jax.experimental.pallas import tpu_sc as plsc

_Q = 4
_D = 128
_B = 16384
_NC = 2
_NS = 16
_NW = _NC * _NS
_RPW = _B // _NW
_VEC = 16
_CH = _RPW // 4


def _sc_body(x_hbm, lt_hbm, out_hbm, xbuf, tbuf, res, sem0, sem1, semt):
    wid = lax.axis_index("s") * _NC + lax.axis_index("c")
    base = wid * _RPW

    def chunk_copy(q, sem):
        return pltpu.make_async_copy(
            x_hbm.at[pl.ds((base + q * _CH) * _D, _CH * _D)],
            xbuf.at[pl.ds(q * _CH * _D, _CH * _D)], sem)

    chunk_copy(0, sem0).start()
    cpt = pltpu.make_async_copy(lt_hbm, tbuf, semt)
    cpt.start()
    for q in range(1, 4):
        chunk_copy(q, sem1).start()

    lanes = lax.iota(jnp.int32, _VEC)
    rows_off = lanes * _D

    _UNROLL = 16

    cpt.wait()
    _BPC = _CH // _VEC

    def blk_body(b, carry):
        @pl.when(b == 0)
        def _():
            chunk_copy(0, sem0).wait()

        @pl.when(b == _BPC)
        def _():
            chunk_copy(1, sem1).wait()

        @pl.when(b == 2 * _BPC)
        def _():
            chunk_copy(2, sem1).wait()

        @pl.when(b == 3 * _BPC)
        def _():
            chunk_copy(3, sem1).wait()

        rows_b = rows_off + b * (_VEC * _D)

        def chunk_steps(u, accs):
            accs = list(accs)
            c0 = (lanes + u * _UNROLL) & (_D - 1)
            cs = [c0, (c0 + 1) & (_D - 1)]
            for t in range(_UNROLL):
                c = cs[t % 2]
                xv = plsc.load_gather(xbuf, [rows_b + c])
                accs[t % 4] = accs[t % 4] + plsc.load_gather(
                    tbuf, [xv * _D + c])
                cs[t % 2] = (c + 2) & (_D - 1)
            return tuple(accs)

        zero = jnp.zeros((_VEC,), jnp.float32)
        accs = lax.fori_loop(0, _D // _UNROLL, chunk_steps, (zero,) * 4)
        res[pl.ds(b * _VEC, _VEC)] = (accs[0] + accs[1]) + (accs[2] + accs[3])
        return carry

    lax.fori_loop(0, _RPW // _VEC, blk_body, 0)

    pltpu.sync_copy(res, out_hbm.at[pl.ds(base, _RPW)])


_sc_call = functools.partial(
    pl.kernel,
    out_type=jax.ShapeDtypeStruct((_B,), jnp.float32),
    mesh=plsc.VectorSubcoreMesh(core_axis_name="c", subcore_axis_name="s"),
    compiler_params=pltpu.CompilerParams(needs_layout_passes=False),
    scratch_types=[
        pltpu.VMEM((_RPW * _D,), jnp.int32),
        pltpu.VMEM((_D * _Q,), jnp.float32),
        pltpu.VMEM((_RPW,), jnp.float32),
        pltpu.SemaphoreType.DMA,
        pltpu.SemaphoreType.DMA,
        pltpu.SemaphoreType.DMA,
    ],
)(_sc_body)


def kernel(x, category_parameters):
    lt = jnp.log(category_parameters * (1.0 - 0.2 * _Q) + 0.2)
    out = _sc_call(x.astype(jnp.int32).reshape(-1),
                   lt.reshape(-1).astype(jnp.float32))
    return lax.stop_gradient(out[:, None])
